# baseline (speedup 1.0000x reference)
import functools

import jax
import jax.numpy as jnp
from jax import lax
from jax.experimental import pallas as pl
from jax.experimental.pallas import tpu as pltpu
from jax.experimental.pallas import tpu_sc as plsc

_NUM_CORES = 2
_NUM_SUBCORES = 16
_NUM_WORKERS = _NUM_CORES * _NUM_SUBCORES
_L = 16

_RADIX_BITS = 11
_NUM_BINS = 1 << _RADIX_BITS
_SHIFTS = (0, _RADIX_BITS, 2 * _RADIX_BITS)
_NS = 4


def _vec(val):
  return lax.full((_L,), val, jnp.int32)


def _lsr(x, k):
  if k == 0:
    return x
  return lax.shift_right_logical(x, _vec(k))


def _to_sortable_bits(f):
  b = plsc.bitcast(f, jnp.int32)
  sgn = lax.shift_right_arithmetic(b, _vec(31))
  flip = lax.bitwise_or(sgn, _vec(-(2**31)))
  return lax.bitwise_xor(b, flip)


@functools.partial(jax.jit, static_argnames=())
def _argsort_rows(x):
  rows, n = x.shape
  assert rows % _NUM_WORKERS == 0 and n % (_L * _NS) == 0
  rows_per_worker = rows // _NUM_WORKERS
  seg_chunks = n // (_L * _NS)
  seg_elems = n // _NS
  seg_shift = seg_elems.bit_length() - 1
  hist_chunks = _NUM_BINS // _L

  mesh = plsc.VectorSubcoreMesh(
      core_axis_name="c", subcore_axis_name="s")

  def body(x_hbm, out_hbm, keyf, ping, pong, hist_a, hist_b):
    cid = lax.axis_index("c")
    sid = lax.axis_index("s")
    wid = sid * _NUM_CORES + cid

    def zero_hist(h):
      def zbody(j, _):
        for s in range(_NS):
          h[pl.ds(s * _NUM_BINS + j * _L, _L)] = _vec(0)
        return 0
      lax.fori_loop(0, hist_chunks, zbody, 0)

    def prefix_hist(h):
      def pbody(j, carry):
        sls = [pl.ds(s * _NUM_BINS + j * _L, _L) for s in range(_NS)]
        vs = [h[sl] for sl in sls]
        tot = vs[0]
        for s in range(1, _NS):
          tot = tot + vs[s]
        base = plsc.cumsum(tot) - tot + carry
        for s in range(_NS):
          h[sls[s]] = base
          if s + 1 < _NS:
            base = base + vs[s]
        return carry + jnp.sum(tot)
      lax.fori_loop(0, hist_chunks, pbody, jnp.int32(-1))

    def pass0_hist():
      def hbody(j, _):
        for s in range(_NS):
          sl = pl.ds((s * seg_chunks + j) * _L, _L)
          u = _to_sortable_bits(keyf[sl])
          keyf[sl] = plsc.bitcast(u, jnp.float32)
          d = lax.bitwise_and(u, _vec(_NUM_BINS - 1))
          occ, last = plsc.scan_count(d)
          plsc.addupdate_scatter(
              hist_a, [lax.bitwise_or(d, _vec(s * _NUM_BINS))], occ, mask=last)
        return 0
      lax.fori_loop(0, seg_chunks, hbody, 0)

    def permute(src, dst, shift, hist_cur, hist_next, next_shift):
      iota = lax.iota(jnp.int32, _L)

      def cbody(j, _):
        for s in range(_NS):
          sl = pl.ds((s * seg_chunks + j) * _L, _L)
          if src is None:
            v_idx = iota + (s * seg_chunks + j) * _L
            u = plsc.bitcast(keyf[sl], jnp.int32)
          else:
            v_idx = src[sl]
            u = plsc.bitcast(plsc.load_gather(keyf, [v_idx]), jnp.int32)
          d = lax.bitwise_and(_lsr(u, shift), _vec(_NUM_BINS - 1))
          occ, last = plsc.scan_count(d)
          dtag = lax.bitwise_or(d, _vec(s * _NUM_BINS))
          base = plsc.load_gather(hist_cur, [dtag])
          pos = base + occ
          plsc.store_scatter(dst, [pos], v_idx)
          plsc.addupdate_scatter(hist_cur, [dtag], occ, mask=last)
          if hist_next is not None:
            d2 = lax.bitwise_and(_lsr(u, next_shift), _vec(_NUM_BINS - 1))
            d2tag = lax.bitwise_or(
                lax.shift_left(_lsr(pos, seg_shift), _vec(_RADIX_BITS)), d2)
            occ2, last2 = plsc.scan_count(d2tag)
            plsc.addupdate_scatter(hist_next, [d2tag], occ2, mask=last2)
        return 0

      lax.fori_loop(0, seg_chunks, cbody, 0)

    def row_body(r, _):
      row = wid * rows_per_worker + r
      pltpu.sync_copy(x_hbm.at[row], keyf)
      zero_hist(hist_a)
      pass0_hist()
      prefix_hist(hist_a)
      zero_hist(hist_b)
      permute(None, ping, _SHIFTS[0], hist_a, hist_b, _SHIFTS[1])
      prefix_hist(hist_b)
      zero_hist(hist_a)
      permute(ping, pong, _SHIFTS[1], hist_b, hist_a, _SHIFTS[2])
      prefix_hist(hist_a)
      permute(pong, ping, _SHIFTS[2], hist_a, None, None)
      pltpu.sync_copy(ping, out_hbm.at[row])
      return 0

    lax.fori_loop(0, rows_per_worker, row_body, 0)

  run = pl.kernel(
      body,
      out_type=jax.ShapeDtypeStruct((rows, n), jnp.int32),
      mesh=mesh,
      compiler_params=pltpu.CompilerParams(needs_layout_passes=False),
      scratch_types=[
          pltpu.VMEM((n,), jnp.float32),
          pltpu.VMEM((n,), jnp.int32),
          pltpu.VMEM((n,), jnp.int32),
          pltpu.VMEM((_NS * _NUM_BINS,), jnp.int32),
          pltpu.VMEM((_NS * _NUM_BINS,), jnp.int32),
      ],
  )
  return run(x)


def kernel(x):
  return _argsort_rows(x)


# speedup vs baseline: 1.4922x; 1.4922x over previous
import functools

import jax
import jax.numpy as jnp
from jax import lax
from jax.experimental import pallas as pl
from jax.experimental.pallas import tpu as pltpu
from jax.experimental.pallas import tpu_sc as plsc

_NUM_CORES = 2
_NUM_SUBCORES = 16
_NUM_WORKERS = _NUM_CORES * _NUM_SUBCORES
_L = 16

_RADIX_BITS = 11
_NUM_BINS = 1 << _RADIX_BITS
_SHIFTS = (0, _RADIX_BITS, 2 * _RADIX_BITS)


def _vec(val):
  return lax.full((_L,), val, jnp.int32)


def _lsr(x, k):
  if k == 0:
    return x
  return lax.shift_right_logical(x, _vec(k))


def _to_sortable_bits(f):
  b = plsc.bitcast(f, jnp.int32)
  sgn = lax.shift_right_arithmetic(b, _vec(31))
  flip = lax.bitwise_or(sgn, _vec(-(2**31)))
  return lax.bitwise_xor(b, flip)


@functools.partial(jax.jit, static_argnames=())
def _argsort_rows(x):
  rows, n = x.shape
  assert rows % _NUM_WORKERS == 0 and n % _L == 0
  rows_per_worker = rows // _NUM_WORKERS
  num_chunks = n // _L
  hist_chunks = _NUM_BINS // _L

  mesh = plsc.VectorSubcoreMesh(
      core_axis_name="c", subcore_axis_name="s")

  def body(x_hbm, out_hbm, keyf, ping, pong, hist0, hist1, hist2):
    cid = lax.axis_index("c")
    sid = lax.axis_index("s")
    wid = sid * _NUM_CORES + cid
    hists = (hist0, hist1, hist2)

    def zero_hists():
      @plsc.parallel_loop(0, hist_chunks, unroll=4)
      def _(j):
        sl = pl.ds(j * _L, _L)
        hist0[sl] = _vec(0)
        hist1[sl] = _vec(0)
        hist2[sl] = _vec(0)

    def transform_and_count():
      @plsc.parallel_loop(0, num_chunks, unroll=4)
      def _(j):
        sl = pl.ds(j * _L, _L)
        u = _to_sortable_bits(keyf[sl])
        keyf[sl] = plsc.bitcast(u, jnp.float32)
        for p in range(3):
          d = lax.bitwise_and(_lsr(u, _SHIFTS[p]), _vec(_NUM_BINS - 1))
          occ, last = plsc.scan_count(d)
          plsc.addupdate_scatter(hists[p], [d], occ, mask=last)

    def prefix_hist(h):
      def pbody(j, carry):
        v = h[pl.ds(j * _L, _L)]
        csum = plsc.cumsum(v)
        h[pl.ds(j * _L, _L)] = csum - v + carry
        return carry + jnp.sum(v)
      lax.fori_loop(0, hist_chunks, pbody, jnp.int32(-1))

    def permute(src, dst, shift, hist_cur):
      iota = lax.iota(jnp.int32, _L)

      def cbody(j, _):
        sl = pl.ds(j * _L, _L)
        if src is None:
          v_idx = iota + j * _L
          u = plsc.bitcast(keyf[sl], jnp.int32)
        else:
          v_idx = src[sl]
          u = plsc.bitcast(plsc.load_gather(keyf, [v_idx]), jnp.int32)
        d = lax.bitwise_and(_lsr(u, shift), _vec(_NUM_BINS - 1))
        occ, last = plsc.scan_count(d)
        base = plsc.load_gather(hist_cur, [d])
        plsc.store_scatter(dst, [base + occ], v_idx)
        plsc.addupdate_scatter(hist_cur, [d], occ, mask=last)
        return 0

      lax.fori_loop(0, num_chunks, cbody, 0)

    def row_body(r, _):
      row = wid * rows_per_worker + r
      pltpu.sync_copy(x_hbm.at[row], keyf)
      zero_hists()
      transform_and_count()
      prefix_hist(hist0)
      permute(None, ping, _SHIFTS[0], hist0)
      prefix_hist(hist1)
      permute(ping, pong, _SHIFTS[1], hist1)
      prefix_hist(hist2)
      permute(pong, ping, _SHIFTS[2], hist2)
      pltpu.sync_copy(ping, out_hbm.at[row])
      return 0

    lax.fori_loop(0, rows_per_worker, row_body, 0)

  run = pl.kernel(
      body,
      out_type=jax.ShapeDtypeStruct((rows, n), jnp.int32),
      mesh=mesh,
      compiler_params=pltpu.CompilerParams(needs_layout_passes=False),
      scratch_types=[
          pltpu.VMEM((n,), jnp.float32),
          pltpu.VMEM((n,), jnp.int32),
          pltpu.VMEM((n,), jnp.int32),
          pltpu.VMEM((_NUM_BINS,), jnp.int32),
          pltpu.VMEM((_NUM_BINS,), jnp.int32),
          pltpu.VMEM((_NUM_BINS,), jnp.int32),
      ],
  )
  return run(x)


def kernel(x):
  return _argsort_rows(x)


# speedup vs baseline: 1.8881x; 1.2653x over previous
import functools

import jax
import jax.numpy as jnp
from jax import lax
from jax.experimental import pallas as pl
from jax.experimental.pallas import tpu as pltpu
from jax.experimental.pallas import tpu_sc as plsc

_NUM_CORES = 2
_NUM_SUBCORES = 16
_NUM_WORKERS = _NUM_CORES * _NUM_SUBCORES
_L = 16

_RADIX_BITS = 11
_NUM_BINS = 1 << _RADIX_BITS
_SHIFTS = (0, _RADIX_BITS, 2 * _RADIX_BITS)


def _vec(val):
  return lax.full((_L,), val, jnp.int32)


def _lsr(x, k):
  if k == 0:
    return x
  return lax.shift_right_logical(x, _vec(k))


def _to_sortable_bits(f):
  b = plsc.bitcast(f, jnp.int32)
  sgn = lax.shift_right_arithmetic(b, _vec(31))
  flip = lax.bitwise_or(sgn, _vec(-(2**31)))
  return lax.bitwise_xor(b, flip)


@functools.partial(jax.jit, static_argnames=())
def _argsort_rows(x):
  rows, n = x.shape
  assert rows % _NUM_WORKERS == 0 and n % _L == 0
  rows_per_worker = rows // _NUM_WORKERS
  num_chunks = n // _L
  hist_chunks = _NUM_BINS // _L

  mesh = plsc.VectorSubcoreMesh(
      core_axis_name="c", subcore_axis_name="s")

  def body(x_hbm, out_hbm, keyf, ping, pong, meta, hist0, hist1, hist2):
    cid = lax.axis_index("c")
    sid = lax.axis_index("s")
    wid = sid * _NUM_CORES + cid
    hists = (hist0, hist1, hist2)

    def zero_hists():
      @plsc.parallel_loop(0, hist_chunks, unroll=4)
      def _(j):
        sl = pl.ds(j * _L, _L)
        hist0[sl] = _vec(0)
        hist1[sl] = _vec(0)
        hist2[sl] = _vec(0)

    def transform_and_count():
      @plsc.parallel_loop(0, num_chunks, unroll=4)
      def _(j):
        sl = pl.ds(j * _L, _L)
        u = _to_sortable_bits(keyf[sl])
        keyf[sl] = plsc.bitcast(u, jnp.float32)
        for p in range(3):
          d = lax.bitwise_and(_lsr(u, _SHIFTS[p]), _vec(_NUM_BINS - 1))
          occ, last = plsc.scan_count(d)
          plsc.addupdate_scatter(hists[p], [d], occ, mask=last)

    def prefix_hist(h):
      def pbody(j, carry):
        v = h[pl.ds(j * _L, _L)]
        csum = plsc.cumsum(v)
        h[pl.ds(j * _L, _L)] = csum - v + carry
        return carry + jnp.sum(v)
      lax.fori_loop(0, hist_chunks, pbody, jnp.int32(-1))

    def permute(src, dst, shift, hist_cur):
      iota = lax.iota(jnp.int32, _L)
      half_chunks = num_chunks // 2

      for half in range(2):
        base_chunk = half * half_chunks

        @plsc.parallel_loop(0, half_chunks, unroll=4)
        def _(j):
          c = j + base_chunk
          sl = pl.ds(c * _L, _L)
          if src is None:
            u = plsc.bitcast(keyf[sl], jnp.int32)
          else:
            u = plsc.bitcast(plsc.load_gather(keyf, [src[sl]]), jnp.int32)
          d = lax.bitwise_and(_lsr(u, shift), _vec(_NUM_BINS - 1))
          occ, last = plsc.scan_count(d)
          lasti = lax.convert_element_type(last, jnp.int32)
          meta[pl.ds(j * _L, _L)] = lax.bitwise_or(
              d,
              lax.bitwise_or(
                  lax.shift_left(occ, _vec(_RADIX_BITS)),
                  lax.shift_left(lasti, _vec(_RADIX_BITS + 5)),
              ),
          )

        def cbody(j, _):
          c = j + base_chunk
          p = meta[pl.ds(j * _L, _L)]
          if src is None:
            v_idx = iota + c * _L
          else:
            v_idx = src[pl.ds(c * _L, _L)]
          d = lax.bitwise_and(p, _vec(_NUM_BINS - 1))
          occ = lax.bitwise_and(_lsr(p, _RADIX_BITS), _vec(31))
          last = lax.ne(_lsr(p, _RADIX_BITS + 5), _vec(0))
          base = plsc.load_gather(hist_cur, [d])
          plsc.store_scatter(dst, [base + occ], v_idx)
          plsc.addupdate_scatter(hist_cur, [d], occ, mask=last)
          return 0

        lax.fori_loop(0, half_chunks, cbody, 0)

    def row_body(r, _):
      row = wid * rows_per_worker + r
      pltpu.sync_copy(x_hbm.at[row], keyf)
      zero_hists()
      transform_and_count()
      prefix_hist(hist0)
      permute(None, ping, _SHIFTS[0], hist0)
      prefix_hist(hist1)
      permute(ping, pong, _SHIFTS[1], hist1)
      prefix_hist(hist2)
      permute(pong, ping, _SHIFTS[2], hist2)
      pltpu.sync_copy(ping, out_hbm.at[row])
      return 0

    lax.fori_loop(0, rows_per_worker, row_body, 0)

  run = pl.kernel(
      body,
      out_type=jax.ShapeDtypeStruct((rows, n), jnp.int32),
      mesh=mesh,
      compiler_params=pltpu.CompilerParams(needs_layout_passes=False),
      scratch_types=[
          pltpu.VMEM((n,), jnp.float32),
          pltpu.VMEM((n,), jnp.int32),
          pltpu.VMEM((n,), jnp.int32),
          pltpu.VMEM((n // 2,), jnp.int32),
          pltpu.VMEM((_NUM_BINS,), jnp.int32),
          pltpu.VMEM((_NUM_BINS,), jnp.int32),
          pltpu.VMEM((_NUM_BINS,), jnp.int32),
      ],
  )
  return run(x)


def kernel(x):
  return _argsort_rows(x)


# speedup vs baseline: 2.5616x; 1.3567x over previous
import functools

import jax
import jax.numpy as jnp
from jax import lax
from jax.experimental import pallas as pl
from jax.experimental.pallas import tpu as pltpu
from jax.experimental.pallas import tpu_sc as plsc

_NUM_CORES = 2
_NUM_SUBCORES = 16
_NUM_WORKERS = _NUM_CORES * _NUM_SUBCORES
_L = 16

_RADIX_BITS = 11
_NUM_BINS = 1 << _RADIX_BITS
_SHIFTS = (0, _RADIX_BITS, 2 * _RADIX_BITS)


def _vec(val):
  return lax.full((_L,), val, jnp.int32)


def _lsr(x, k):
  if k == 0:
    return x
  return lax.shift_right_logical(x, _vec(k))


def _to_sortable_bits(f):
  b = plsc.bitcast(f, jnp.int32)
  sgn = lax.shift_right_arithmetic(b, _vec(31))
  flip = lax.bitwise_or(sgn, _vec(-(2**31)))
  return lax.bitwise_xor(b, flip)


@functools.partial(jax.jit, static_argnames=())
def _argsort_rows(x):
  rows, n = x.shape
  assert rows % _NUM_WORKERS == 0 and n % _L == 0
  rows_per_worker = rows // _NUM_WORKERS
  num_chunks = n // _L
  hist_chunks = _NUM_BINS // _L

  mesh = plsc.VectorSubcoreMesh(
      core_axis_name="c", subcore_axis_name="s")

  def body(x_hbm, out_hbm, keyf, ping, pong, meta, hist0, hist1, hist2):
    cid = lax.axis_index("c")
    sid = lax.axis_index("s")
    wid = sid * _NUM_CORES + cid
    hists = (hist0, hist1, hist2)

    def zero_hists():
      @plsc.parallel_loop(0, hist_chunks, unroll=4)
      def _(j):
        sl = pl.ds(j * _L, _L)
        hist0[sl] = _vec(0)
        hist1[sl] = _vec(0)
        hist2[sl] = _vec(0)

    def transform_and_count():
      @plsc.parallel_loop(0, num_chunks, unroll=4)
      def _(j):
        sl = pl.ds(j * _L, _L)
        u = _to_sortable_bits(keyf[sl])
        keyf[sl] = plsc.bitcast(u, jnp.float32)
        for p in range(3):
          d = lax.bitwise_and(_lsr(u, _SHIFTS[p]), _vec(_NUM_BINS - 1))
          occ, last = plsc.scan_count(d)
          plsc.addupdate_scatter(hists[p], [d], occ, mask=last)

    def prefix_hist(h):
      def pbody(j, carry):
        v = h[pl.ds(j * _L, _L)]
        csum = plsc.cumsum(v)
        h[pl.ds(j * _L, _L)] = csum - v + carry
        return carry + jnp.sum(v)
      lax.fori_loop(0, hist_chunks, pbody, jnp.int32(-1))

    def permute(src, dst, shift, hist_cur):
      iota = lax.iota(jnp.int32, _L)
      half_chunks = num_chunks // 2

      for half in range(2):
        base_chunk = half * half_chunks

        @plsc.parallel_loop(0, half_chunks, unroll=4)
        def _(j):
          c = j + base_chunk
          sl = pl.ds(c * _L, _L)
          if src is None:
            u = plsc.bitcast(keyf[sl], jnp.int32)
          else:
            u = plsc.bitcast(plsc.load_gather(keyf, [src[sl]]), jnp.int32)
          d = lax.bitwise_and(_lsr(u, shift), _vec(_NUM_BINS - 1))
          occ, last = plsc.scan_count(d)
          lasti = lax.convert_element_type(last, jnp.int32)
          meta[pl.ds(j * _L, _L)] = lax.bitwise_or(
              d,
              lax.bitwise_or(
                  lax.shift_left(occ, _vec(_RADIX_BITS)),
                  lax.shift_left(lasti, _vec(_RADIX_BITS + 5)),
              ),
          )

        group = 4

        def cbody(g, _):
          regs = []
          for k in range(group):
            j = g * group + k
            c = j + base_chunk
            p = meta[pl.ds(j * _L, _L)]
            if src is None:
              v_idx = iota + c * _L
            else:
              v_idx = src[pl.ds(c * _L, _L)]
            d = lax.bitwise_and(p, _vec(_NUM_BINS - 1))
            occ = lax.bitwise_and(_lsr(p, _RADIX_BITS), _vec(31))
            last = lax.ne(_lsr(p, _RADIX_BITS + 5), _vec(0))
            regs.append((d, occ, last, v_idx))
          for d, occ, last, v_idx in regs:
            base = plsc.load_gather(hist_cur, [d])
            plsc.store_scatter(dst, [base + occ], v_idx)
            plsc.addupdate_scatter(hist_cur, [d], occ, mask=last)
          return 0

        lax.fori_loop(0, half_chunks // group, cbody, 0)

    def row_body(r, _):
      row = wid * rows_per_worker + r
      pltpu.sync_copy(x_hbm.at[row], keyf)
      zero_hists()
      transform_and_count()
      prefix_hist(hist0)
      permute(None, ping, _SHIFTS[0], hist0)
      prefix_hist(hist1)
      permute(ping, pong, _SHIFTS[1], hist1)
      prefix_hist(hist2)
      permute(pong, ping, _SHIFTS[2], hist2)
      pltpu.sync_copy(ping, out_hbm.at[row])
      return 0

    lax.fori_loop(0, rows_per_worker, row_body, 0)

  run = pl.kernel(
      body,
      out_type=jax.ShapeDtypeStruct((rows, n), jnp.int32),
      mesh=mesh,
      compiler_params=pltpu.CompilerParams(needs_layout_passes=False),
      scratch_types=[
          pltpu.VMEM((n,), jnp.float32),
          pltpu.VMEM((n,), jnp.int32),
          pltpu.VMEM((n,), jnp.int32),
          pltpu.VMEM((n // 2,), jnp.int32),
          pltpu.VMEM((_NUM_BINS,), jnp.int32),
          pltpu.VMEM((_NUM_BINS,), jnp.int32),
          pltpu.VMEM((_NUM_BINS,), jnp.int32),
      ],
  )
  return run(x)


def kernel(x):
  return _argsort_rows(x)


# speedup vs baseline: 2.7074x; 1.0569x over previous
import functools

import jax
import jax.numpy as jnp
from jax import lax
from jax.experimental import pallas as pl
from jax.experimental.pallas import tpu as pltpu
from jax.experimental.pallas import tpu_sc as plsc

_NUM_CORES = 2
_NUM_SUBCORES = 16
_NUM_WORKERS = _NUM_CORES * _NUM_SUBCORES
_L = 16

_RADIX_BITS = 11
_NUM_BINS = 1 << _RADIX_BITS
_SHIFTS = (0, _RADIX_BITS, 2 * _RADIX_BITS)


def _vec(val):
  return lax.full((_L,), val, jnp.int32)


def _lsr(x, k):
  if k == 0:
    return x
  return lax.shift_right_logical(x, _vec(k))


def _to_sortable_bits(f):
  b = plsc.bitcast(f, jnp.int32)
  sgn = lax.shift_right_arithmetic(b, _vec(31))
  flip = lax.bitwise_or(sgn, _vec(-(2**31)))
  return lax.bitwise_xor(b, flip)


@functools.partial(jax.jit, static_argnames=())
def _argsort_rows(x):
  rows, n = x.shape
  assert rows % _NUM_WORKERS == 0 and n % _L == 0
  rows_per_worker = rows // _NUM_WORKERS
  num_chunks = n // _L
  hist_chunks = _NUM_BINS // _L

  mesh = plsc.VectorSubcoreMesh(
      core_axis_name="c", subcore_axis_name="s")

  def body(x_hbm, out_hbm, keyf, ping, pong, meta, hist0, hist1, hist2):
    cid = lax.axis_index("c")
    sid = lax.axis_index("s")
    wid = sid * _NUM_CORES + cid
    hists = (hist0, hist1, hist2)

    def zero_hists():
      @plsc.parallel_loop(0, hist_chunks, unroll=4)
      def _(j):
        sl = pl.ds(j * _L, _L)
        hist0[sl] = _vec(0)
        hist1[sl] = _vec(0)
        hist2[sl] = _vec(0)

    def transform_and_count():
      @plsc.parallel_loop(0, num_chunks, unroll=4)
      def _(j):
        sl = pl.ds(j * _L, _L)
        u = _to_sortable_bits(keyf[sl])
        keyf[sl] = plsc.bitcast(u, jnp.float32)
        for p in range(3):
          d = lax.bitwise_and(_lsr(u, _SHIFTS[p]), _vec(_NUM_BINS - 1))
          occ, last = plsc.scan_count(d)
          plsc.addupdate_scatter(hists[p], [d], occ, mask=last)

    def prefix_hist(h):
      def pbody(j, carry):
        v = h[pl.ds(j * _L, _L)]
        csum = plsc.cumsum(v)
        h[pl.ds(j * _L, _L)] = csum - v + carry
        return carry + jnp.sum(v)
      lax.fori_loop(0, hist_chunks, pbody, jnp.int32(-1))

    def permute(src, dst, shift, hist_cur):
      iota = lax.iota(jnp.int32, _L)
      half_chunks = num_chunks // 2

      for half in range(2):
        base_chunk = half * half_chunks

        @plsc.parallel_loop(0, half_chunks, unroll=4)
        def _(j):
          c = j + base_chunk
          sl = pl.ds(c * _L, _L)
          if src is None:
            u = plsc.bitcast(keyf[sl], jnp.int32)
          else:
            u = plsc.bitcast(plsc.load_gather(keyf, [src[sl]]), jnp.int32)
          d = lax.bitwise_and(_lsr(u, shift), _vec(_NUM_BINS - 1))
          occ, last = plsc.scan_count(d)
          lasti = lax.convert_element_type(last, jnp.int32)
          meta[pl.ds(j * _L, _L)] = lax.bitwise_or(
              d,
              lax.bitwise_or(
                  lax.shift_left(occ, _vec(_RADIX_BITS)),
                  lax.shift_left(lasti, _vec(_RADIX_BITS + 5)),
              ),
          )

        group = 8

        def cbody(g, _):
          regs = []
          for k in range(group):
            j = g * group + k
            c = j + base_chunk
            p = meta[pl.ds(j * _L, _L)]
            if src is None:
              v_idx = iota + c * _L
            else:
              v_idx = src[pl.ds(c * _L, _L)]
            d = lax.bitwise_and(p, _vec(_NUM_BINS - 1))
            occ = lax.bitwise_and(_lsr(p, _RADIX_BITS), _vec(31))
            last = lax.ne(_lsr(p, _RADIX_BITS + 5), _vec(0))
            regs.append((d, occ, last, v_idx))
          for d, occ, last, v_idx in regs:
            base = plsc.load_gather(hist_cur, [d])
            plsc.store_scatter(dst, [base + occ], v_idx)
            plsc.addupdate_scatter(hist_cur, [d], occ, mask=last)
          return 0

        lax.fori_loop(0, half_chunks // group, cbody, 0)

    def row_body(r, _):
      row = wid * rows_per_worker + r
      pltpu.sync_copy(x_hbm.at[row], keyf)
      zero_hists()
      transform_and_count()
      prefix_hist(hist0)
      permute(None, ping, _SHIFTS[0], hist0)
      prefix_hist(hist1)
      permute(ping, pong, _SHIFTS[1], hist1)
      prefix_hist(hist2)
      permute(pong, ping, _SHIFTS[2], hist2)
      pltpu.sync_copy(ping, out_hbm.at[row])
      return 0

    lax.fori_loop(0, rows_per_worker, row_body, 0)

  run = pl.kernel(
      body,
      out_type=jax.ShapeDtypeStruct((rows, n), jnp.int32),
      mesh=mesh,
      compiler_params=pltpu.CompilerParams(needs_layout_passes=False),
      scratch_types=[
          pltpu.VMEM((n,), jnp.float32),
          pltpu.VMEM((n,), jnp.int32),
          pltpu.VMEM((n,), jnp.int32),
          pltpu.VMEM((n // 2,), jnp.int32),
          pltpu.VMEM((_NUM_BINS,), jnp.int32),
          pltpu.VMEM((_NUM_BINS,), jnp.int32),
          pltpu.VMEM((_NUM_BINS,), jnp.int32),
      ],
  )
  return run(x)


def kernel(x):
  return _argsort_rows(x)


# speedup vs baseline: 2.7201x; 1.0047x over previous
import functools

import jax
import jax.numpy as jnp
from jax import lax
from jax.experimental import pallas as pl
from jax.experimental.pallas import tpu as pltpu
from jax.experimental.pallas import tpu_sc as plsc

_NUM_CORES = 2
_NUM_SUBCORES = 16
_NUM_WORKERS = _NUM_CORES * _NUM_SUBCORES
_L = 16

_RADIX_BITS = 11
_NUM_BINS = 1 << _RADIX_BITS
_SHIFTS = (0, _RADIX_BITS, 2 * _RADIX_BITS)
_NSEG = 4


def _vec(val):
  return lax.full((_L,), val, jnp.int32)


def _lsr(x, k):
  if k == 0:
    return x
  return lax.shift_right_logical(x, _vec(k))


def _to_sortable_bits(f):
  b = plsc.bitcast(f, jnp.int32)
  sgn = lax.shift_right_arithmetic(b, _vec(31))
  flip = lax.bitwise_or(sgn, _vec(-(2**31)))
  return lax.bitwise_xor(b, flip)


@functools.partial(jax.jit, static_argnames=())
def _argsort_rows(x):
  rows, n = x.shape
  assert rows % _NUM_WORKERS == 0 and n % (_L * _NSEG) == 0
  rows_per_worker = rows // _NUM_WORKERS
  seg_chunks = n // (_L * _NSEG)
  hist_chunks = _NUM_BINS // _L

  mesh = plsc.VectorSubcoreMesh(
      core_axis_name="c", subcore_axis_name="s")

  def body(x_hbm, out_hbm, keyf, ping, pong, h0, h1, h2, h3):
    cid = lax.axis_index("c")
    sid = lax.axis_index("s")
    wid = sid * _NUM_CORES + cid
    hsegs = (h0, h1, h2, h3)

    def zero_hists():
      @plsc.parallel_loop(0, hist_chunks, unroll=4)
      def _(j):
        sl = pl.ds(j * _L, _L)
        for h in hsegs:
          h[sl] = _vec(0)

    def prefix_hists():
      def pbody(j, carry):
        sl = pl.ds(j * _L, _L)
        vs = [h[sl] for h in hsegs]
        tot = vs[0]
        for s in range(1, _NSEG):
          tot = tot + vs[s]
        base = plsc.cumsum(tot) - tot + carry
        for s in range(_NSEG):
          hsegs[s][sl] = base
          if s + 1 < _NSEG:
            base = base + vs[s]
        return carry + jnp.sum(tot)
      lax.fori_loop(0, hist_chunks, pbody, jnp.int32(-1))

    def transform_and_count0():
      @plsc.parallel_loop(0, seg_chunks, unroll=2)
      def _(j):
        for s in range(_NSEG):
          sl = pl.ds((s * seg_chunks + j) * _L, _L)
          u = _to_sortable_bits(keyf[sl])
          keyf[sl] = plsc.bitcast(u, jnp.float32)
          d = lax.bitwise_and(u, _vec(_NUM_BINS - 1))
          occ, last = plsc.scan_count(d)
          plsc.addupdate_scatter(hsegs[s], [d], occ, mask=last)

    def count_sweep(src, shift):
      @plsc.parallel_loop(0, seg_chunks, unroll=2)
      def _(j):
        for s in range(_NSEG):
          sl = pl.ds((s * seg_chunks + j) * _L, _L)
          u = plsc.bitcast(plsc.load_gather(keyf, [src[sl]]), jnp.int32)
          d = lax.bitwise_and(_lsr(u, shift), _vec(_NUM_BINS - 1))
          occ, last = plsc.scan_count(d)
          plsc.addupdate_scatter(hsegs[s], [d], occ, mask=last)

    def permute(src, dst, shift):
      iota = lax.iota(jnp.int32, _L)
      rounds_per_iter = 2

      def cbody(g, _):
        work = []
        for r in range(rounds_per_iter):
          j = g * rounds_per_iter + r
          for s in range(_NSEG):
            c = s * seg_chunks + j
            sl = pl.ds(c * _L, _L)
            if src is None:
              v_idx = iota + c * _L
              u = plsc.bitcast(keyf[sl], jnp.int32)
            else:
              v_idx = src[sl]
              u = plsc.bitcast(plsc.load_gather(keyf, [v_idx]), jnp.int32)
            d = lax.bitwise_and(_lsr(u, shift), _vec(_NUM_BINS - 1))
            occ, last = plsc.scan_count(d)
            work.append((s, d, occ, last, v_idx))
        for s, d, occ, last, v_idx in work:
          base = plsc.load_gather(hsegs[s], [d])
          pos = base + occ
          plsc.store_scatter(hsegs[s], [d], pos, mask=last)
          plsc.store_scatter(dst, [pos], v_idx)
        return 0

      lax.fori_loop(0, seg_chunks // rounds_per_iter, cbody, 0)

    def row_body(r, _):
      row = wid * rows_per_worker + r
      pltpu.sync_copy(x_hbm.at[row], keyf)
      zero_hists()
      transform_and_count0()
      prefix_hists()
      permute(None, ping, _SHIFTS[0])
      zero_hists()
      count_sweep(ping, _SHIFTS[1])
      prefix_hists()
      permute(ping, pong, _SHIFTS[1])
      zero_hists()
      count_sweep(pong, _SHIFTS[2])
      prefix_hists()
      permute(pong, ping, _SHIFTS[2])
      pltpu.sync_copy(ping, out_hbm.at[row])
      return 0

    lax.fori_loop(0, rows_per_worker, row_body, 0)

  run = pl.kernel(
      body,
      out_type=jax.ShapeDtypeStruct((rows, n), jnp.int32),
      mesh=mesh,
      compiler_params=pltpu.CompilerParams(needs_layout_passes=False),
      scratch_types=[
          pltpu.VMEM((n,), jnp.float32),
          pltpu.VMEM((n,), jnp.int32),
          pltpu.VMEM((n,), jnp.int32),
          pltpu.VMEM((_NUM_BINS,), jnp.int32),
          pltpu.VMEM((_NUM_BINS,), jnp.int32),
          pltpu.VMEM((_NUM_BINS,), jnp.int32),
          pltpu.VMEM((_NUM_BINS,), jnp.int32),
      ],
  )
  return run(x)


def kernel(x):
  return _argsort_rows(x)
